# baseline (speedup 1.0000x reference)
import dataclasses

import jax
import jax.numpy as jnp
from jax import lax
from jax.experimental import pallas as pl
from jax.experimental.pallas import tpu as pltpu
from jax.experimental.pallas import tpu_sc as plsc

_N = 10000
_IN = 128
_HEADS = 8
_HID = 64
_NG = 256
_LANES = 16
_RS = 64
_NR = 160
_NPAD = _RS * _NR
_NTILES = 32
_RPT = _NR // _NTILES
_K = 32
_CAP = 3072
_OFFS_LEN = 192


def _proj_mm_body(x_ref, wl_ref, bl_ref, wr_ref, br_ref, ol_ref, or_ref):
    x = x_ref[...]
    ol_ref[...] = jnp.dot(x, wl_ref[...], preferred_element_type=jnp.float32) + bl_ref[...]
    or_ref[...] = jnp.dot(x, wr_ref[...], preferred_element_type=jnp.float32) + br_ref[...]


def _proj(x, Wl, bl, Wr, br, bm):
    n, d = x.shape
    dout = Wl.shape[1]
    grid = n // bm
    return pl.pallas_call(
        _proj_mm_body,
        grid=(grid,),
        in_specs=[
            pl.BlockSpec((bm, d), lambda i: (i, 0)),
            pl.BlockSpec((d, dout), lambda i: (0, 0)),
            pl.BlockSpec((1, dout), lambda i: (0, 0)),
            pl.BlockSpec((d, dout), lambda i: (0, 0)),
            pl.BlockSpec((1, dout), lambda i: (0, 0)),
        ],
        out_specs=[
            pl.BlockSpec((bm, dout), lambda i: (i, 0)),
            pl.BlockSpec((bm, dout), lambda i: (i, 0)),
        ],
        out_shape=[jax.ShapeDtypeStruct((n, dout), jnp.float32)] * 2,
    )(x, Wl, bl.reshape(1, dout), Wr, br.reshape(1, dout))


def _gat_sc(xl, xr_pad, srcs, dsts, offs, att, bias, heads, do_act, ncomp=None):
    D = xl.shape[1]
    NV = D // _LANES
    NC = NV if ncomp is None else ncomp
    mesh = plsc.VectorSubcoreMesh(core_axis_name="c", subcore_axis_name="s")

    def body(xl_hbm, xr_hbm, src_hbm, dst_hbm, offs_hbm, att_hbm, bias_hbm, out_hbm,
             offs_v, att_v, bias_v, idxs_v, dsts_v, rows0_v, rows1_v,
             xr_v, acc_v, den_v, red_v, sem0, sem1):
        def _sread(ref, i):
            return ref[pl.ds(i, _LANES)][0]

        tid = lax.axis_index("s") * 2 + lax.axis_index("c")
        pltpu.sync_copy(offs_hbm, offs_v)
        pltpu.sync_copy(att_hbm, att_v)
        pltpu.sync_copy(bias_hbm, bias_v)
        lane = lax.iota(jnp.int32, _LANES)
        idx_lo = jnp.bitwise_and(lane, 7)
        idx_hi = idx_lo + 8

        def _gather(c, rows_ref, sem):
            return pltpu.make_async_copy(
                xl_hbm.at[idxs_v.at[pl.ds(c * _K, _K)]], rows_ref, sem)

        @pl.loop(0, _RPT)
        def _range_loop(ri):
            r = tid * _RPT + ri
            lo = r * _RS
            eo = _sread(offs_v, r)
            e1 = _sread(offs_v, r + 1)
            base0 = pl.multiple_of(jnp.bitwise_and(eo, -8), 8)
            nch = (e1 - base0 + _K - 1) // _K

            pltpu.sync_copy(src_hbm.at[pl.ds(base0, _CAP)], idxs_v)
            pltpu.sync_copy(dst_hbm.at[pl.ds(base0, _CAP)],
                            dsts_v.at[pl.ds(0, _CAP)])

            @pl.when(nch > 0)
            def _():
                _gather(0, rows0_v, sem0).start()

            @pl.when(nch > 1)
            def _():
                _gather(1, rows1_v, sem1).start()

            pltpu.sync_copy(xr_hbm.at[pl.ds(lo, _RS)], xr_v)

            @pl.loop(0, _RS)
            def _zero(dd):
                zero = jnp.zeros((_LANES,), jnp.float32)
                for v in range(NV):
                    acc_v[dd, pl.ds(v * _LANES, _LANES)] = zero
                den_v[dd, :] = zero

            def process(c, rows_ref, sem):
                _gather(c, rows_ref, sem).wait()
                b = c * _K
                jlo = jnp.maximum(eo - base0 - b, 0)
                jhi = jnp.minimum(e1 - base0 - b, _K)

                def edge(j, carry2):
                    dl = _sread(dsts_v, b + j) - lo
                    nacc = min(NC, 4)
                    laccs = [jnp.zeros((_LANES,), jnp.float32)] * nacc
                    for v in range(NC):
                        sl = pl.ds(v * _LANES, _LANES)
                        z = rows_ref[j, sl] + xr_v[dl, sl]
                        m = jnp.maximum(z, 0.2 * z)
                        laccs[v % nacc] = laccs[v % nacc] + m * att_v[sl]
                    while len(laccs) > 1:
                        laccs = [a + b2 for a, b2 in zip(laccs[::2], laccs[1::2])]
                    lacc = laccs[0]
                    if heads == 8:
                        red_v[...] = lacc
                        l2 = (plsc.load_gather(red_v, [idx_lo])
                              + plsc.load_gather(red_v, [idx_hi]))
                        a = jnp.exp(l2)
                    else:
                        s = jnp.sum(lacc)
                        a = jnp.exp(jnp.broadcast_to(s, (_LANES,)))
                    for v in range(NC):
                        sl = pl.ds(v * _LANES, _LANES)
                        plsc.addupdate(acc_v.at[dl, sl], a * rows_ref[j, sl])
                    plsc.addupdate(den_v.at[dl], a)
                    return carry2

                lax.fori_loop(jlo, jhi, edge, None)

                @pl.when(c + 2 < nch)
                def _():
                    _gather(c + 2, rows_ref, sem).start()

            def pair(i, carry):
                c0 = 2 * i

                @pl.when(c0 < nch)
                def _():
                    process(c0, rows0_v, sem0)

                @pl.when(c0 + 1 < nch)
                def _():
                    process(c0 + 1, rows1_v, sem1)

                return carry

            lax.fori_loop(0, (nch + 1) // 2, pair, None)

            @pl.loop(0, _RS)
            def _norm(dd):
                dv = den_v[dd, :]
                for v in range(NC):
                    sl = pl.ds(v * _LANES, _LANES)
                    o = acc_v[dd, sl] / dv + bias_v[sl]
                    if do_act:
                        o = jnp.where(o > 0.0, o, jnp.exp(o) - 1.0)
                    acc_v[dd, sl] = o

            pltpu.sync_copy(acc_v, out_hbm.at[pl.ds(lo, _RS)])

    cp = pltpu.CompilerParams()
    if "needs_layout_passes" in pltpu.CompilerParams.__dataclass_fields__:
        cp = dataclasses.replace(cp, needs_layout_passes=False)
    f = pl.kernel(
        body,
        out_type=jax.ShapeDtypeStruct((_NPAD, D), jnp.float32),
        mesh=mesh,
        compiler_params=cp,
        scratch_types=[
            pltpu.VMEM((_OFFS_LEN,), jnp.int32),
            pltpu.VMEM((D,), jnp.float32),
            pltpu.VMEM((D,), jnp.float32),
            pltpu.VMEM((_CAP,), jnp.int32),
            pltpu.VMEM((_CAP + _LANES,), jnp.int32),
            pltpu.VMEM((_K, D), jnp.float32),
            pltpu.VMEM((_K, D), jnp.float32),
            pltpu.VMEM((_RS, D), jnp.float32),
            pltpu.VMEM((_RS, D), jnp.float32),
            pltpu.VMEM((_RS, _LANES), jnp.float32),
            pltpu.VMEM((_LANES,), jnp.float32),
            pltpu.SemaphoreType.DMA,
            pltpu.SemaphoreType.DMA,
        ],
    )
    return f(xl, xr_pad, srcs, dsts, offs, att, bias)


def _pool_body(h_ref, b_ref, wc_ref, bc_ref, o_ref, acc_ref):
    i = pl.program_id(0)

    @pl.when(i == 0)
    def _():
        acc_ref[...] = jnp.zeros_like(acc_ref)

    bvals = b_ref[...].reshape(1, -1)
    oh = (bvals == lax.broadcasted_iota(jnp.int32, (_NG, bvals.shape[1]), 0)
          ).astype(jnp.float32)
    acc_ref[...] += jnp.dot(oh, h_ref[...], preferred_element_type=jnp.float32)

    @pl.when(i == pl.num_programs(0) - 1)
    def _():
        cnt = acc_ref[:, _HID:_HID + 1]
        pooled = acc_ref[:, :_HID] / jnp.maximum(cnt, 1.0)
        o_ref[...] = jnp.dot(pooled, wc_ref[...],
                             preferred_element_type=jnp.float32) + bc_ref[...]


def _pool(hcat, batch3, Wc, bc, bm):
    n = hcat.shape[0]
    dcat = hcat.shape[1]
    grid = n // bm
    return pl.pallas_call(
        _pool_body,
        grid=(grid,),
        in_specs=[
            pl.BlockSpec((bm, dcat), lambda i: (i, 0)),
            pl.BlockSpec((1, 1, bm), lambda i: (i, 0, 0)),
            pl.BlockSpec((_HID, 12), lambda i: (0, 0)),
            pl.BlockSpec((1, 12), lambda i: (0, 0)),
        ],
        out_specs=pl.BlockSpec((_NG, 12), lambda i: (0, 0)),
        out_shape=jax.ShapeDtypeStruct((_NG, 12), jnp.float32),
        scratch_shapes=[pltpu.VMEM((_NG, dcat), jnp.float32)],
    )(hcat, batch3, Wc, bc.reshape(1, 12))


def kernel(x, edge_index, batch, Wl1, bl1, Wr1, br1, att1, bias1,
           Wl2, bl2, Wr2, br2, att2, bias2, Wc, bc):
    n = x.shape[0]
    loops_idx = jnp.arange(n, dtype=edge_index.dtype)
    src = jnp.concatenate([edge_index[0], loops_idx])
    dst = jnp.concatenate([edge_index[1], loops_idx])
    e_tot = src.shape[0]
    perm = jnp.argsort(dst)
    srcs = src[perm].astype(jnp.int32)
    dsts = dst[perm].astype(jnp.int32)
    bounds = jnp.arange(_OFFS_LEN, dtype=jnp.int32) * _RS
    offs = jnp.searchsorted(dsts, bounds).astype(jnp.int32)
    pad = _CAP + 64
    srcs_p = jnp.concatenate([srcs, jnp.zeros((pad,), jnp.int32)])
    dsts_p = jnp.concatenate([dsts, jnp.zeros((pad,), jnp.int32)])

    d1 = _HEADS * _HID
    Wl1p = Wl1.reshape(_IN, _HEADS, _HID).transpose(0, 2, 1).reshape(_IN, d1)
    Wr1p = Wr1.reshape(_IN, _HEADS, _HID).transpose(0, 2, 1).reshape(_IN, d1)
    bl1p = bl1.reshape(_HEADS, _HID).T.reshape(d1)
    br1p = br1.reshape(_HEADS, _HID).T.reshape(d1)
    att1t = att1.T.reshape(d1)
    bias1t = bias1.reshape(_HEADS, _HID).T.reshape(d1)
    xl1, xr1 = _proj(x, Wl1p, bl1p, Wr1p, br1p, bm=1000)
    xr1p = jnp.concatenate([xr1, jnp.zeros((_NPAD - n, d1), jnp.float32)], axis=0)
    h1 = _gat_sc(xl1, xr1p, srcs_p, dsts_p, offs, att1t, bias1t,
                 heads=_HEADS, do_act=True)

    zpad = jnp.zeros((d1, _HID), jnp.float32)
    Wl2p = jnp.concatenate(
        [Wl2.reshape(_HEADS, _HID, _HID).transpose(1, 0, 2).reshape(d1, _HID), zpad], axis=1)
    Wr2p = jnp.concatenate(
        [Wr2.reshape(_HEADS, _HID, _HID).transpose(1, 0, 2).reshape(d1, _HID), zpad], axis=1)
    z64 = jnp.zeros((_HID,), jnp.float32)
    bl2p = jnp.concatenate([bl2, z64])
    br2p = jnp.concatenate([br2, z64])
    att2p = jnp.concatenate([att2.reshape(_HID), z64])
    bias2p = jnp.concatenate([bias2, z64])
    xl2, xr2 = _proj(h1, Wl2p, bl2p, Wr2p, br2p, bm=1024)
    h2 = _gat_sc(xl2, xr2, srcs_p, dsts_p, offs, att2p, bias2p,
                 heads=1, do_act=False, ncomp=_HID // _LANES)

    hcat = jnp.concatenate([h2[:n, :_HID], jnp.ones((n, 8), jnp.float32)], axis=1)
    batch3 = batch.astype(jnp.int32).reshape(10, 1, 1000)
    return _pool(hcat, batch3, Wc, bc, bm=1000)


# speedup vs baseline: 1.5124x; 1.5124x over previous
import dataclasses

import jax
import jax.numpy as jnp
from jax import lax
from jax.experimental import pallas as pl
from jax.experimental.pallas import tpu as pltpu
from jax.experimental.pallas import tpu_sc as plsc

_N = 10000
_IN = 128
_HEADS = 8
_HID = 64
_NG = 256
_LANES = 16
_RS = 64
_NR = 160
_NPAD = _RS * _NR
_NTILES = 32
_RPT = _NR // _NTILES
_K = 32
_CAP = 3072
_OFFS_LEN = 192


def _proj_mm_body(x_ref, wl_ref, bl_ref, wr_ref, br_ref, ol_ref, or_ref):
    x = x_ref[...]
    ol_ref[...] = jnp.dot(x, wl_ref[...], preferred_element_type=jnp.float32) + bl_ref[...]
    or_ref[...] = jnp.dot(x, wr_ref[...], preferred_element_type=jnp.float32) + br_ref[...]


def _proj(x, Wl, bl, Wr, br, bm):
    n, d = x.shape
    dout = Wl.shape[1]
    grid = n // bm
    return pl.pallas_call(
        _proj_mm_body,
        grid=(grid,),
        in_specs=[
            pl.BlockSpec((bm, d), lambda i: (i, 0)),
            pl.BlockSpec((d, dout), lambda i: (0, 0)),
            pl.BlockSpec((1, dout), lambda i: (0, 0)),
            pl.BlockSpec((d, dout), lambda i: (0, 0)),
            pl.BlockSpec((1, dout), lambda i: (0, 0)),
        ],
        out_specs=[
            pl.BlockSpec((bm, dout), lambda i: (i, 0)),
            pl.BlockSpec((bm, dout), lambda i: (i, 0)),
        ],
        out_shape=[jax.ShapeDtypeStruct((n, dout), jnp.float32)] * 2,
    )(x, Wl, bl.reshape(1, dout), Wr, br.reshape(1, dout))


def _gat_sc(xl, xr_pad, srcs, dsts, offs, att, bias, heads, do_act, ncomp=None):
    D = xl.shape[1]
    NV = D // _LANES
    NC = NV if ncomp is None else ncomp
    mesh = plsc.VectorSubcoreMesh(core_axis_name="c", subcore_axis_name="s")

    def body(xl_hbm, xr_hbm, src_hbm, dst_hbm, offs_hbm, att_hbm, bias_hbm, out_hbm,
             offs_v, att_v, bias_v, idxs_v, dsts_v, rows0_v, rows1_v,
             xr_v, acc_v, den_v, red_v, sem0, sem1):
        def _sread(ref, i):
            return ref[pl.ds(i, _LANES)][0]

        tid = lax.axis_index("s") * 2 + lax.axis_index("c")
        pltpu.sync_copy(offs_hbm, offs_v)
        pltpu.sync_copy(att_hbm, att_v)
        pltpu.sync_copy(bias_hbm, bias_v)
        lane = lax.iota(jnp.int32, _LANES)
        idx_lo = jnp.bitwise_and(lane, 7)
        idx_hi = idx_lo + 8

        def _gather(c, rows_ref, sem):
            return pltpu.make_async_copy(
                xl_hbm.at[idxs_v.at[pl.ds(c * _K, _K)]], rows_ref, sem)

        @pl.loop(0, _RPT)
        def _range_loop(ri):
            r = tid * _RPT + ri
            lo = r * _RS
            eo = _sread(offs_v, r)
            e1 = _sread(offs_v, r + 1)
            base0 = pl.multiple_of(jnp.bitwise_and(eo, -8), 8)
            nch = (e1 - base0 + _K - 1) // _K

            pltpu.sync_copy(src_hbm.at[pl.ds(base0, _CAP)], idxs_v)
            pltpu.sync_copy(dst_hbm.at[pl.ds(base0, _CAP)],
                            dsts_v.at[pl.ds(0, _CAP)])

            @pl.when(nch > 0)
            def _():
                _gather(0, rows0_v, sem0).start()

            @pl.when(nch > 1)
            def _():
                _gather(1, rows1_v, sem1).start()

            pltpu.sync_copy(xr_hbm.at[pl.ds(lo, _RS)], xr_v)

            @pl.loop(0, _RS)
            def _zero(dd):
                zero = jnp.zeros((_LANES,), jnp.float32)
                for v in range(NV):
                    acc_v[dd, pl.ds(v * _LANES, _LANES)] = zero
                den_v[dd, :] = zero

            def process(c, rows_ref, sem):
                _gather(c, rows_ref, sem).wait()
                b = c * _K
                jlo = jnp.maximum(eo - base0 - b, 0)
                jhi = jnp.minimum(e1 - base0 - b, _K)

                def edge(j, carry2):
                    dl = _sread(dsts_v, b + j) - lo
                    nacc = min(NC, 4)
                    laccs = [jnp.zeros((_LANES,), jnp.float32)] * nacc
                    for v in range(NC):
                        sl = pl.ds(v * _LANES, _LANES)
                        z = rows_ref[j, sl] + xr_v[dl, sl]
                        m = jnp.maximum(z, 0.2 * z)
                        laccs[v % nacc] = laccs[v % nacc] + m * att_v[sl]
                    while len(laccs) > 1:
                        laccs = [a + b2 for a, b2 in zip(laccs[::2], laccs[1::2])]
                    lacc = laccs[0]
                    if heads == 8:
                        red_v[...] = lacc
                        l2 = (plsc.load_gather(red_v, [idx_lo])
                              + plsc.load_gather(red_v, [idx_hi]))
                        a = jnp.exp(l2)
                    else:
                        s = jnp.sum(lacc)
                        a = jnp.exp(jnp.broadcast_to(s, (_LANES,)))
                    prods = [a * rows_ref[j, pl.ds(v * _LANES, _LANES)]
                             for v in range(NC)]
                    for v in range(NC):
                        plsc.addupdate(acc_v.at[dl, pl.ds(v * _LANES, _LANES)],
                                       prods[v])
                    plsc.addupdate(den_v.at[dl], a)
                    return carry2

                lax.fori_loop(jlo, jhi, edge, None)

                @pl.when(c + 2 < nch)
                def _():
                    _gather(c + 2, rows_ref, sem).start()

            def pair(i, carry):
                c0 = 2 * i

                @pl.when(c0 < nch)
                def _():
                    process(c0, rows0_v, sem0)

                @pl.when(c0 + 1 < nch)
                def _():
                    process(c0 + 1, rows1_v, sem1)

                return carry

            lax.fori_loop(0, (nch + 1) // 2, pair, None)

            @pl.loop(0, _RS)
            def _norm(dd):
                dv = den_v[dd, :]
                for v in range(NC):
                    sl = pl.ds(v * _LANES, _LANES)
                    o = acc_v[dd, sl] / dv + bias_v[sl]
                    if do_act:
                        o = jnp.where(o > 0.0, o, jnp.exp(o) - 1.0)
                    acc_v[dd, sl] = o

            pltpu.sync_copy(acc_v, out_hbm.at[pl.ds(lo, _RS)])

    cp = pltpu.CompilerParams()
    if "needs_layout_passes" in pltpu.CompilerParams.__dataclass_fields__:
        cp = dataclasses.replace(cp, needs_layout_passes=False)
    f = pl.kernel(
        body,
        out_type=jax.ShapeDtypeStruct((_NPAD, D), jnp.float32),
        mesh=mesh,
        compiler_params=cp,
        scratch_types=[
            pltpu.VMEM((_OFFS_LEN,), jnp.int32),
            pltpu.VMEM((D,), jnp.float32),
            pltpu.VMEM((D,), jnp.float32),
            pltpu.VMEM((_CAP,), jnp.int32),
            pltpu.VMEM((_CAP + _LANES,), jnp.int32),
            pltpu.VMEM((_K, D), jnp.float32),
            pltpu.VMEM((_K, D), jnp.float32),
            pltpu.VMEM((_RS, D), jnp.float32),
            pltpu.VMEM((_RS, D), jnp.float32),
            pltpu.VMEM((_RS, _LANES), jnp.float32),
            pltpu.VMEM((_LANES,), jnp.float32),
            pltpu.SemaphoreType.DMA,
            pltpu.SemaphoreType.DMA,
        ],
    )
    return f(xl, xr_pad, srcs, dsts, offs, att, bias)


def _pool_body(h_ref, b_ref, wc_ref, bc_ref, o_ref, acc_ref):
    i = pl.program_id(0)

    @pl.when(i == 0)
    def _():
        acc_ref[...] = jnp.zeros_like(acc_ref)

    bvals = b_ref[...].reshape(1, -1)
    oh = (bvals == lax.broadcasted_iota(jnp.int32, (_NG, bvals.shape[1]), 0)
          ).astype(jnp.float32)
    acc_ref[...] += jnp.dot(oh, h_ref[...], preferred_element_type=jnp.float32)

    @pl.when(i == pl.num_programs(0) - 1)
    def _():
        cnt = acc_ref[:, _HID:_HID + 1]
        pooled = acc_ref[:, :_HID] / jnp.maximum(cnt, 1.0)
        o_ref[...] = jnp.dot(pooled, wc_ref[...],
                             preferred_element_type=jnp.float32) + bc_ref[...]


def _pool(hcat, batch3, Wc, bc, bm):
    n = hcat.shape[0]
    dcat = hcat.shape[1]
    grid = n // bm
    return pl.pallas_call(
        _pool_body,
        grid=(grid,),
        in_specs=[
            pl.BlockSpec((bm, dcat), lambda i: (i, 0)),
            pl.BlockSpec((1, 1, bm), lambda i: (i, 0, 0)),
            pl.BlockSpec((_HID, 12), lambda i: (0, 0)),
            pl.BlockSpec((1, 12), lambda i: (0, 0)),
        ],
        out_specs=pl.BlockSpec((_NG, 12), lambda i: (0, 0)),
        out_shape=jax.ShapeDtypeStruct((_NG, 12), jnp.float32),
        scratch_shapes=[pltpu.VMEM((_NG, dcat), jnp.float32)],
    )(hcat, batch3, Wc, bc.reshape(1, 12))


def kernel(x, edge_index, batch, Wl1, bl1, Wr1, br1, att1, bias1,
           Wl2, bl2, Wr2, br2, att2, bias2, Wc, bc):
    n = x.shape[0]
    loops_idx = jnp.arange(n, dtype=edge_index.dtype)
    src = jnp.concatenate([edge_index[0], loops_idx])
    dst = jnp.concatenate([edge_index[1], loops_idx])
    e_tot = src.shape[0]
    perm = jnp.argsort(dst)
    srcs = src[perm].astype(jnp.int32)
    dsts = dst[perm].astype(jnp.int32)
    bounds = jnp.arange(_OFFS_LEN, dtype=jnp.int32) * _RS
    offs = jnp.searchsorted(dsts, bounds).astype(jnp.int32)
    pad = _CAP + 64
    srcs_p = jnp.concatenate([srcs, jnp.zeros((pad,), jnp.int32)])
    dsts_p = jnp.concatenate([dsts, jnp.zeros((pad,), jnp.int32)])

    d1 = _HEADS * _HID
    Wl1p = Wl1.reshape(_IN, _HEADS, _HID).transpose(0, 2, 1).reshape(_IN, d1)
    Wr1p = Wr1.reshape(_IN, _HEADS, _HID).transpose(0, 2, 1).reshape(_IN, d1)
    bl1p = bl1.reshape(_HEADS, _HID).T.reshape(d1)
    br1p = br1.reshape(_HEADS, _HID).T.reshape(d1)
    att1t = att1.T.reshape(d1)
    bias1t = bias1.reshape(_HEADS, _HID).T.reshape(d1)
    xl1, xr1 = _proj(x, Wl1p, bl1p, Wr1p, br1p, bm=1000)
    xr1p = jnp.concatenate([xr1, jnp.zeros((_NPAD - n, d1), jnp.float32)], axis=0)
    h1 = _gat_sc(xl1, xr1p, srcs_p, dsts_p, offs, att1t, bias1t,
                 heads=_HEADS, do_act=True)

    zpad = jnp.zeros((d1, _HID), jnp.float32)
    Wl2p = jnp.concatenate(
        [Wl2.reshape(_HEADS, _HID, _HID).transpose(1, 0, 2).reshape(d1, _HID), zpad], axis=1)
    Wr2p = jnp.concatenate(
        [Wr2.reshape(_HEADS, _HID, _HID).transpose(1, 0, 2).reshape(d1, _HID), zpad], axis=1)
    z64 = jnp.zeros((_HID,), jnp.float32)
    bl2p = jnp.concatenate([bl2, z64])
    br2p = jnp.concatenate([br2, z64])
    att2p = jnp.concatenate([att2.reshape(_HID), z64])
    bias2p = jnp.concatenate([bias2, z64])
    xl2, xr2 = _proj(h1, Wl2p, bl2p, Wr2p, br2p, bm=1024)
    h2 = _gat_sc(xl2, xr2, srcs_p, dsts_p, offs, att2p, bias2p,
                 heads=1, do_act=False, ncomp=_HID // _LANES)

    hcat = jnp.concatenate([h2[:n, :_HID], jnp.ones((n, 8), jnp.float32)], axis=1)
    batch3 = batch.astype(jnp.int32).reshape(10, 1, 1000)
    return _pool(hcat, batch3, Wc, bc, bm=1000)


# speedup vs baseline: 1.6098x; 1.0644x over previous
import dataclasses

import jax
import jax.numpy as jnp
from jax import lax
from jax.experimental import pallas as pl
from jax.experimental.pallas import tpu as pltpu
from jax.experimental.pallas import tpu_sc as plsc

_N = 10000
_IN = 128
_HEADS = 8
_HID = 64
_NG = 256
_LANES = 16
_RS = 64
_NR = 160
_NPAD = _RS * _NR
_NTILES = 32
_RPT = _NR // _NTILES
_K = 32
_CAP = 3072
_OFFS_LEN = 192


def _proj_mm_body(x_ref, wl_ref, bl_ref, wr_ref, br_ref, ol_ref, or_ref):
    x = x_ref[...]
    ol_ref[...] = jnp.dot(x, wl_ref[...], preferred_element_type=jnp.float32) + bl_ref[...]
    or_ref[...] = jnp.dot(x, wr_ref[...], preferred_element_type=jnp.float32) + br_ref[...]


def _proj(x, Wl, bl, Wr, br, bm):
    n, d = x.shape
    dout = Wl.shape[1]
    grid = n // bm
    return pl.pallas_call(
        _proj_mm_body,
        grid=(grid,),
        in_specs=[
            pl.BlockSpec((bm, d), lambda i: (i, 0)),
            pl.BlockSpec((d, dout), lambda i: (0, 0)),
            pl.BlockSpec((1, dout), lambda i: (0, 0)),
            pl.BlockSpec((d, dout), lambda i: (0, 0)),
            pl.BlockSpec((1, dout), lambda i: (0, 0)),
        ],
        out_specs=[
            pl.BlockSpec((bm, dout), lambda i: (i, 0)),
            pl.BlockSpec((bm, dout), lambda i: (i, 0)),
        ],
        out_shape=[jax.ShapeDtypeStruct((n, dout), jnp.float32)] * 2,
    )(x, Wl, bl.reshape(1, dout), Wr, br.reshape(1, dout))


def _gat_sc(xl, xr_pad, srcs, dsts, offs, att, bias, heads, do_act, ncomp=None):
    D = xl.shape[1]
    NV = D // _LANES
    NC = NV if ncomp is None else ncomp
    mesh = plsc.VectorSubcoreMesh(core_axis_name="c", subcore_axis_name="s")

    def body(xl_hbm, xr_hbm, src_hbm, dst_hbm, offs_hbm, att_hbm, bias_hbm, out_hbm,
             offs_v, att_v, bias_v, idxs_v, dsts_v, rows0_v, rows1_v,
             xr_v, acc_v, den_v, red_v, sem0, sem1):
        def _sread(ref, i):
            return ref[pl.ds(i, _LANES)][0]

        tid = lax.axis_index("s") * 2 + lax.axis_index("c")
        pltpu.sync_copy(offs_hbm, offs_v)
        pltpu.sync_copy(att_hbm, att_v)
        pltpu.sync_copy(bias_hbm, bias_v)
        lane = lax.iota(jnp.int32, _LANES)
        idx_lo = jnp.bitwise_and(lane, 7)
        idx_hi = idx_lo + 8

        def _gather(c, rows_ref, sem):
            return pltpu.make_async_copy(
                xl_hbm.at[idxs_v.at[pl.ds(c * _K, _K)]], rows_ref, sem)

        @pl.loop(0, _RPT)
        def _range_loop(ri):
            r = tid * _RPT + ri
            lo = r * _RS
            eo = _sread(offs_v, r)
            e1 = _sread(offs_v, r + 1)
            base0 = pl.multiple_of(jnp.bitwise_and(eo, -8), 8)
            nch = (e1 - base0 + _K - 1) // _K

            pltpu.sync_copy(src_hbm.at[pl.ds(base0, _CAP)], idxs_v)
            pltpu.sync_copy(dst_hbm.at[pl.ds(base0, _CAP)],
                            dsts_v.at[pl.ds(0, _CAP)])

            @pl.when(nch > 0)
            def _():
                _gather(0, rows0_v, sem0).start()

            @pl.when(nch > 1)
            def _():
                _gather(1, rows1_v, sem1).start()

            pltpu.sync_copy(xr_hbm.at[pl.ds(lo, _RS)], xr_v)

            @pl.loop(0, _RS)
            def _zero(dd):
                zero = jnp.zeros((_LANES,), jnp.float32)
                for v in range(NV):
                    acc_v[dd, pl.ds(v * _LANES, _LANES)] = zero
                den_v[dd, :] = zero

            def process(c, rows_ref, sem):
                _gather(c, rows_ref, sem).wait()
                b = c * _K
                jlo = jnp.maximum(eo - base0 - b, 0)
                jhi = jnp.minimum(e1 - base0 - b, _K)

                def edge(j, carry2):
                    dl = _sread(dsts_v, b + j) - lo
                    nacc = min(NC, 4)
                    laccs = [jnp.zeros((_LANES,), jnp.float32)] * nacc
                    for v in range(NC):
                        sl = pl.ds(v * _LANES, _LANES)
                        z = rows_ref[j, sl] + xr_v[dl, sl]
                        m = jnp.maximum(z, 0.2 * z)
                        laccs[v % nacc] = laccs[v % nacc] + m * att_v[sl]
                    while len(laccs) > 1:
                        laccs = [a + b2 for a, b2 in zip(laccs[::2], laccs[1::2])]
                    lacc = laccs[0]
                    if heads == 8:
                        red_v[...] = lacc
                        l2 = (plsc.load_gather(red_v, [idx_lo])
                              + plsc.load_gather(red_v, [idx_hi]))
                        a = jnp.exp(l2)
                    else:
                        s = jnp.sum(lacc)
                        a = jnp.exp(jnp.broadcast_to(s, (_LANES,)))
                    prods = [a * rows_ref[j, pl.ds(v * _LANES, _LANES)]
                             for v in range(NC)]
                    for v in range(NC):
                        plsc.addupdate(acc_v.at[dl, pl.ds(v * _LANES, _LANES)],
                                       prods[v])
                    plsc.addupdate(den_v.at[dl], a)
                    return carry2

                lax.fori_loop(jlo, jhi, edge, None)

                @pl.when(c + 2 < nch)
                def _():
                    _gather(c + 2, rows_ref, sem).start()

            def pair(i, carry):
                c0 = 2 * i

                @pl.when(c0 < nch)
                def _():
                    process(c0, rows0_v, sem0)

                @pl.when(c0 + 1 < nch)
                def _():
                    process(c0 + 1, rows1_v, sem1)

                return carry

            lax.fori_loop(0, (nch + 1) // 2, pair, None)

            @pl.loop(0, _RS)
            def _norm(dd):
                dv = den_v[dd, :]
                for v in range(NC):
                    sl = pl.ds(v * _LANES, _LANES)
                    o = acc_v[dd, sl] / dv + bias_v[sl]
                    if do_act:
                        o = jnp.where(o > 0.0, o, jnp.exp(o) - 1.0)
                    acc_v[dd, sl] = o

            pltpu.sync_copy(acc_v, out_hbm.at[pl.ds(lo, _RS)])

    cp = pltpu.CompilerParams()
    if "needs_layout_passes" in pltpu.CompilerParams.__dataclass_fields__:
        cp = dataclasses.replace(cp, needs_layout_passes=False)
    f = pl.kernel(
        body,
        out_type=jax.ShapeDtypeStruct((_NPAD, D), jnp.float32),
        mesh=mesh,
        compiler_params=cp,
        scratch_types=[
            pltpu.VMEM((_OFFS_LEN,), jnp.int32),
            pltpu.VMEM((D,), jnp.float32),
            pltpu.VMEM((D,), jnp.float32),
            pltpu.VMEM((_CAP,), jnp.int32),
            pltpu.VMEM((_CAP + _LANES,), jnp.int32),
            pltpu.VMEM((_K, D), jnp.float32),
            pltpu.VMEM((_K, D), jnp.float32),
            pltpu.VMEM((_RS, D), jnp.float32),
            pltpu.VMEM((_RS, D), jnp.float32),
            pltpu.VMEM((_RS, _LANES), jnp.float32),
            pltpu.VMEM((_LANES,), jnp.float32),
            pltpu.SemaphoreType.DMA,
            pltpu.SemaphoreType.DMA,
        ],
    )
    return f(xl, xr_pad, srcs, dsts, offs, att, bias)


def _pool_body(h_ref, b_ref, wc_ref, bc_ref, o_ref, acc_ref):
    i = pl.program_id(0)

    @pl.when(i == 0)
    def _():
        acc_ref[...] = jnp.zeros_like(acc_ref)

    bvals = b_ref[...].reshape(1, -1)
    oh = (bvals == lax.broadcasted_iota(jnp.int32, (_NG, bvals.shape[1]), 0)
          ).astype(jnp.float32)
    acc_ref[...] += jnp.dot(oh, h_ref[...], preferred_element_type=jnp.float32)

    @pl.when(i == pl.num_programs(0) - 1)
    def _():
        cnt = acc_ref[:, _HID:_HID + 1]
        pooled = acc_ref[:, :_HID] / jnp.maximum(cnt, 1.0)
        o_ref[...] = jnp.dot(pooled, wc_ref[...],
                             preferred_element_type=jnp.float32) + bc_ref[...]


def _pool(hcat, batch3, Wc, bc, bm):
    n = hcat.shape[0]
    dcat = hcat.shape[1]
    grid = n // bm
    return pl.pallas_call(
        _pool_body,
        grid=(grid,),
        in_specs=[
            pl.BlockSpec((bm, dcat), lambda i: (i, 0)),
            pl.BlockSpec((1, 1, bm), lambda i: (i, 0, 0)),
            pl.BlockSpec((_HID, 12), lambda i: (0, 0)),
            pl.BlockSpec((1, 12), lambda i: (0, 0)),
        ],
        out_specs=pl.BlockSpec((_NG, 12), lambda i: (0, 0)),
        out_shape=jax.ShapeDtypeStruct((_NG, 12), jnp.float32),
        scratch_shapes=[pltpu.VMEM((_NG, dcat), jnp.float32)],
    )(hcat, batch3, Wc, bc.reshape(1, 12))


def kernel(x, edge_index, batch, Wl1, bl1, Wr1, br1, att1, bias1,
           Wl2, bl2, Wr2, br2, att2, bias2, Wc, bc):
    n = x.shape[0]
    loops_idx = jnp.arange(n, dtype=edge_index.dtype)
    src = jnp.concatenate([edge_index[0], loops_idx])
    dst = jnp.concatenate([edge_index[1], loops_idx])
    e_tot = src.shape[0]
    packed = (dst.astype(jnp.uint32) << 18) | jnp.arange(e_tot, dtype=jnp.uint32)
    sp = jnp.sort(packed)
    dsts = (sp >> 18).astype(jnp.int32)
    perm = (sp & jnp.uint32(0x3FFFF)).astype(jnp.int32)
    srcs = src[perm].astype(jnp.int32)
    bounds = jnp.arange(_OFFS_LEN, dtype=jnp.int32) * _RS
    offs = jnp.searchsorted(dsts, bounds).astype(jnp.int32)
    pad = _CAP + 64
    srcs_p = jnp.concatenate([srcs, jnp.zeros((pad,), jnp.int32)])
    dsts_p = jnp.concatenate([dsts, jnp.zeros((pad,), jnp.int32)])

    d1 = _HEADS * _HID
    Wl1p = Wl1.reshape(_IN, _HEADS, _HID).transpose(0, 2, 1).reshape(_IN, d1)
    Wr1p = Wr1.reshape(_IN, _HEADS, _HID).transpose(0, 2, 1).reshape(_IN, d1)
    bl1p = bl1.reshape(_HEADS, _HID).T.reshape(d1)
    br1p = br1.reshape(_HEADS, _HID).T.reshape(d1)
    att1t = att1.T.reshape(d1)
    bias1t = bias1.reshape(_HEADS, _HID).T.reshape(d1)
    xl1, xr1 = _proj(x, Wl1p, bl1p, Wr1p, br1p, bm=1000)
    xr1p = jnp.concatenate([xr1, jnp.zeros((_NPAD - n, d1), jnp.float32)], axis=0)
    h1 = _gat_sc(xl1, xr1p, srcs_p, dsts_p, offs, att1t, bias1t,
                 heads=_HEADS, do_act=True)

    zpad = jnp.zeros((d1, _HID), jnp.float32)
    Wl2p = jnp.concatenate(
        [Wl2.reshape(_HEADS, _HID, _HID).transpose(1, 0, 2).reshape(d1, _HID), zpad], axis=1)
    Wr2p = jnp.concatenate(
        [Wr2.reshape(_HEADS, _HID, _HID).transpose(1, 0, 2).reshape(d1, _HID), zpad], axis=1)
    z64 = jnp.zeros((_HID,), jnp.float32)
    bl2p = jnp.concatenate([bl2, z64])
    br2p = jnp.concatenate([br2, z64])
    att2p = jnp.concatenate([att2.reshape(_HID), z64])
    bias2p = jnp.concatenate([bias2, z64])
    xl2, xr2 = _proj(h1, Wl2p, bl2p, Wr2p, br2p, bm=1024)
    h2 = _gat_sc(xl2, xr2, srcs_p, dsts_p, offs, att2p, bias2p,
                 heads=1, do_act=False, ncomp=_HID // _LANES)

    hcat = jnp.concatenate([h2[:n, :_HID], jnp.ones((n, 8), jnp.float32)], axis=1)
    batch3 = batch.astype(jnp.int32).reshape(10, 1, 1000)
    return _pool(hcat, batch3, Wc, bc, bm=1000)


# speedup vs baseline: 1.8239x; 1.1330x over previous
import dataclasses

import jax
import jax.numpy as jnp
from jax import lax
from jax.experimental import pallas as pl
from jax.experimental.pallas import tpu as pltpu
from jax.experimental.pallas import tpu_sc as plsc

_N = 10000
_IN = 128
_HEADS = 8
_HID = 64
_NG = 256
_LANES = 16
_RS = 64
_NR = 160
_NPAD = _RS * _NR
_NTILES = 32
_RPT = _NR // _NTILES
_K = 32
_CAP = 3072
_OFFS_LEN = 192


def _proj_mm_body(x_ref, wl_ref, bl_ref, wr_ref, br_ref, ol_ref, or_ref):
    x = x_ref[...]
    ol_ref[...] = jnp.dot(x, wl_ref[...], preferred_element_type=jnp.float32) + bl_ref[...]
    or_ref[...] = jnp.dot(x, wr_ref[...], preferred_element_type=jnp.float32) + br_ref[...]


def _proj(x, Wl, bl, Wr, br, bm):
    n, d = x.shape
    dout = Wl.shape[1]
    grid = n // bm
    return pl.pallas_call(
        _proj_mm_body,
        grid=(grid,),
        in_specs=[
            pl.BlockSpec((bm, d), lambda i: (i, 0)),
            pl.BlockSpec((d, dout), lambda i: (0, 0)),
            pl.BlockSpec((1, dout), lambda i: (0, 0)),
            pl.BlockSpec((d, dout), lambda i: (0, 0)),
            pl.BlockSpec((1, dout), lambda i: (0, 0)),
        ],
        out_specs=[
            pl.BlockSpec((bm, dout), lambda i: (i, 0)),
            pl.BlockSpec((bm, dout), lambda i: (i, 0)),
        ],
        out_shape=[jax.ShapeDtypeStruct((n, dout), jnp.float32)] * 2,
    )(x, Wl, bl.reshape(1, dout), Wr, br.reshape(1, dout))


def _gat_sc(xl, xr_pad, srcs, dsts, offs, att, bias, heads, do_act, ncomp=None):
    D = xl.shape[1]
    NV = D // _LANES
    NC = NV if ncomp is None else ncomp
    mesh = plsc.VectorSubcoreMesh(core_axis_name="c", subcore_axis_name="s")

    def body(xl_hbm, xr_hbm, src_hbm, dst_hbm, offs_hbm, att_hbm, bias_hbm, out_hbm,
             offs_v, att_v, bias_v, idxs_v, dsts_v, rows0_v, rows1_v,
             xr_v, acc_v, den_v, red_v, sem0, sem1):
        def _sread(ref, i):
            return ref[pl.ds(i, _LANES)][0]

        tid = lax.axis_index("s") * 2 + lax.axis_index("c")
        pltpu.sync_copy(offs_hbm, offs_v)
        pltpu.sync_copy(att_hbm, att_v)
        pltpu.sync_copy(bias_hbm, bias_v)
        lane = lax.iota(jnp.int32, _LANES)
        idx_lo = jnp.bitwise_and(lane, 7)
        idx_hi = idx_lo + 8

        def _gather(c, rows_ref, sem):
            return pltpu.make_async_copy(
                xl_hbm.at[idxs_v.at[pl.ds(c * _K, _K)]], rows_ref, sem)

        @pl.loop(0, _RPT)
        def _range_loop(ri):
            r = tid * _RPT + ri
            lo = r * _RS
            eo = _sread(offs_v, r)
            e1 = _sread(offs_v, r + 1)
            base0 = pl.multiple_of(jnp.bitwise_and(eo, -8), 8)
            nch = (e1 - base0 + _K - 1) // _K

            pltpu.sync_copy(src_hbm.at[pl.ds(base0, _CAP)], idxs_v)
            pltpu.sync_copy(dst_hbm.at[pl.ds(base0, _CAP)],
                            dsts_v.at[pl.ds(0, _CAP)])

            @pl.when(nch > 0)
            def _():
                _gather(0, rows0_v, sem0).start()

            @pl.when(nch > 1)
            def _():
                _gather(1, rows1_v, sem1).start()

            pltpu.sync_copy(xr_hbm.at[pl.ds(lo, _RS)], xr_v)

            @pl.loop(0, _RS)
            def _zero(dd):
                zero = jnp.zeros((_LANES,), jnp.float32)
                for v in range(NV):
                    acc_v[dd, pl.ds(v * _LANES, _LANES)] = zero
                den_v[dd, :] = zero

            def process(c, rows_ref, sem):
                _gather(c, rows_ref, sem).wait()
                b = c * _K
                jlo = jnp.maximum(eo - base0 - b, 0)
                jhi = jnp.minimum(e1 - base0 - b, _K)

                def _fold_exp(lacc):
                    if heads == 8:
                        red_v[...] = lacc
                        l2 = (plsc.load_gather(red_v, [idx_lo])
                              + plsc.load_gather(red_v, [idx_hi]))
                        return jnp.exp(l2)
                    s = jnp.sum(lacc)
                    return jnp.exp(jnp.broadcast_to(s, (_LANES,)))

                def _accum(j, dl, a):
                    prods = [a * rows_ref[j, pl.ds(v * _LANES, _LANES)]
                             for v in range(NC)]
                    for v in range(NC):
                        plsc.addupdate(acc_v.at[dl, pl.ds(v * _LANES, _LANES)],
                                       prods[v])
                    plsc.addupdate(den_v.at[dl], a)

                nacc = min(NC, 4)

                def _logit1(j, dl):
                    laccs = [jnp.zeros((_LANES,), jnp.float32)] * nacc
                    for v in range(NC):
                        sl = pl.ds(v * _LANES, _LANES)
                        z = rows_ref[j, sl] + xr_v[dl, sl]
                        m = jnp.maximum(z, 0.2 * z)
                        laccs[v % nacc] = laccs[v % nacc] + m * att_v[sl]
                    while len(laccs) > 1:
                        laccs = [a + b2 for a, b2 in zip(laccs[::2], laccs[1::2])]
                    return laccs[0]

                def edge_pair(k, carry2):
                    j0 = jlo + 2 * k
                    j1 = j0 + 1
                    dl0 = _sread(dsts_v, b + j0) - lo
                    dl1 = _sread(dsts_v, b + j1) - lo
                    na = min(NC, 2)
                    l0 = [jnp.zeros((_LANES,), jnp.float32)] * na
                    l1 = [jnp.zeros((_LANES,), jnp.float32)] * na
                    for v in range(NC):
                        sl = pl.ds(v * _LANES, _LANES)
                        av = att_v[sl]
                        z0 = rows_ref[j0, sl] + xr_v[dl0, sl]
                        z1 = rows_ref[j1, sl] + xr_v[dl1, sl]
                        m0 = jnp.maximum(z0, 0.2 * z0)
                        m1 = jnp.maximum(z1, 0.2 * z1)
                        l0[v % na] = l0[v % na] + m0 * av
                        l1[v % na] = l1[v % na] + m1 * av
                    a0 = _fold_exp(sum(l0[1:], l0[0]))
                    a1 = _fold_exp(sum(l1[1:], l1[0]))
                    _accum(j0, dl0, a0)
                    _accum(j1, dl1, a1)
                    return carry2

                ne = jhi - jlo
                npairs = ne // 2
                lax.fori_loop(0, npairs, edge_pair, None)

                @pl.when(ne > 2 * npairs)
                def _():
                    j = jhi - 1
                    dl = _sread(dsts_v, b + j) - lo
                    _accum(j, dl, _fold_exp(_logit1(j, dl)))

                @pl.when(c + 2 < nch)
                def _():
                    _gather(c + 2, rows_ref, sem).start()

            def pair(i, carry):
                c0 = 2 * i

                @pl.when(c0 < nch)
                def _():
                    process(c0, rows0_v, sem0)

                @pl.when(c0 + 1 < nch)
                def _():
                    process(c0 + 1, rows1_v, sem1)

                return carry

            lax.fori_loop(0, (nch + 1) // 2, pair, None)

            @pl.loop(0, _RS)
            def _norm(dd):
                dv = den_v[dd, :]
                for v in range(NC):
                    sl = pl.ds(v * _LANES, _LANES)
                    o = acc_v[dd, sl] / dv + bias_v[sl]
                    if do_act:
                        o = jnp.where(o > 0.0, o, jnp.exp(o) - 1.0)
                    acc_v[dd, sl] = o

            pltpu.sync_copy(acc_v, out_hbm.at[pl.ds(lo, _RS)])

    cp = pltpu.CompilerParams()
    if "needs_layout_passes" in pltpu.CompilerParams.__dataclass_fields__:
        cp = dataclasses.replace(cp, needs_layout_passes=False)
    f = pl.kernel(
        body,
        out_type=jax.ShapeDtypeStruct((_NPAD, D), jnp.float32),
        mesh=mesh,
        compiler_params=cp,
        scratch_types=[
            pltpu.VMEM((_OFFS_LEN,), jnp.int32),
            pltpu.VMEM((D,), jnp.float32),
            pltpu.VMEM((D,), jnp.float32),
            pltpu.VMEM((_CAP,), jnp.int32),
            pltpu.VMEM((_CAP + _LANES,), jnp.int32),
            pltpu.VMEM((_K, D), jnp.float32),
            pltpu.VMEM((_K, D), jnp.float32),
            pltpu.VMEM((_RS, D), jnp.float32),
            pltpu.VMEM((_RS, D), jnp.float32),
            pltpu.VMEM((_RS, _LANES), jnp.float32),
            pltpu.VMEM((_LANES,), jnp.float32),
            pltpu.SemaphoreType.DMA,
            pltpu.SemaphoreType.DMA,
        ],
    )
    return f(xl, xr_pad, srcs, dsts, offs, att, bias)


def _pool_body(h_ref, b_ref, wc_ref, bc_ref, o_ref, acc_ref):
    i = pl.program_id(0)

    @pl.when(i == 0)
    def _():
        acc_ref[...] = jnp.zeros_like(acc_ref)

    bvals = b_ref[...].reshape(1, -1)
    oh = (bvals == lax.broadcasted_iota(jnp.int32, (_NG, bvals.shape[1]), 0)
          ).astype(jnp.float32)
    acc_ref[...] += jnp.dot(oh, h_ref[...], preferred_element_type=jnp.float32)

    @pl.when(i == pl.num_programs(0) - 1)
    def _():
        cnt = acc_ref[:, _HID:_HID + 1]
        pooled = acc_ref[:, :_HID] / jnp.maximum(cnt, 1.0)
        o_ref[...] = jnp.dot(pooled, wc_ref[...],
                             preferred_element_type=jnp.float32) + bc_ref[...]


def _pool(hcat, batch3, Wc, bc, bm):
    n = hcat.shape[0]
    dcat = hcat.shape[1]
    grid = n // bm
    return pl.pallas_call(
        _pool_body,
        grid=(grid,),
        in_specs=[
            pl.BlockSpec((bm, dcat), lambda i: (i, 0)),
            pl.BlockSpec((1, 1, bm), lambda i: (i, 0, 0)),
            pl.BlockSpec((_HID, 12), lambda i: (0, 0)),
            pl.BlockSpec((1, 12), lambda i: (0, 0)),
        ],
        out_specs=pl.BlockSpec((_NG, 12), lambda i: (0, 0)),
        out_shape=jax.ShapeDtypeStruct((_NG, 12), jnp.float32),
        scratch_shapes=[pltpu.VMEM((_NG, dcat), jnp.float32)],
    )(hcat, batch3, Wc, bc.reshape(1, 12))


def kernel(x, edge_index, batch, Wl1, bl1, Wr1, br1, att1, bias1,
           Wl2, bl2, Wr2, br2, att2, bias2, Wc, bc):
    n = x.shape[0]
    loops_idx = jnp.arange(n, dtype=edge_index.dtype)
    src = jnp.concatenate([edge_index[0], loops_idx])
    dst = jnp.concatenate([edge_index[1], loops_idx])
    e_tot = src.shape[0]
    packed = (dst.astype(jnp.uint32) << 18) | jnp.arange(e_tot, dtype=jnp.uint32)
    sp = jnp.sort(packed)
    dsts = (sp >> 18).astype(jnp.int32)
    perm = (sp & jnp.uint32(0x3FFFF)).astype(jnp.int32)
    srcs = src[perm].astype(jnp.int32)
    bounds = jnp.arange(_OFFS_LEN, dtype=jnp.int32) * _RS
    offs = jnp.searchsorted(dsts, bounds).astype(jnp.int32)
    pad = _CAP + 64
    srcs_p = jnp.concatenate([srcs, jnp.zeros((pad,), jnp.int32)])
    dsts_p = jnp.concatenate([dsts, jnp.zeros((pad,), jnp.int32)])

    d1 = _HEADS * _HID
    Wl1p = Wl1.reshape(_IN, _HEADS, _HID).transpose(0, 2, 1).reshape(_IN, d1)
    Wr1p = Wr1.reshape(_IN, _HEADS, _HID).transpose(0, 2, 1).reshape(_IN, d1)
    bl1p = bl1.reshape(_HEADS, _HID).T.reshape(d1)
    br1p = br1.reshape(_HEADS, _HID).T.reshape(d1)
    att1t = att1.T.reshape(d1)
    bias1t = bias1.reshape(_HEADS, _HID).T.reshape(d1)
    xl1, xr1 = _proj(x, Wl1p, bl1p, Wr1p, br1p, bm=1000)
    xr1p = jnp.concatenate([xr1, jnp.zeros((_NPAD - n, d1), jnp.float32)], axis=0)
    h1 = _gat_sc(xl1, xr1p, srcs_p, dsts_p, offs, att1t, bias1t,
                 heads=_HEADS, do_act=True)

    zpad = jnp.zeros((d1, _HID), jnp.float32)
    Wl2p = jnp.concatenate(
        [Wl2.reshape(_HEADS, _HID, _HID).transpose(1, 0, 2).reshape(d1, _HID), zpad], axis=1)
    Wr2p = jnp.concatenate(
        [Wr2.reshape(_HEADS, _HID, _HID).transpose(1, 0, 2).reshape(d1, _HID), zpad], axis=1)
    z64 = jnp.zeros((_HID,), jnp.float32)
    bl2p = jnp.concatenate([bl2, z64])
    br2p = jnp.concatenate([br2, z64])
    att2p = jnp.concatenate([att2.reshape(_HID), z64])
    bias2p = jnp.concatenate([bias2, z64])
    xl2, xr2 = _proj(h1, Wl2p, bl2p, Wr2p, br2p, bm=1024)
    h2 = _gat_sc(xl2, xr2, srcs_p, dsts_p, offs, att2p, bias2p,
                 heads=1, do_act=False, ncomp=_HID // _LANES)

    hcat = jnp.concatenate([h2[:n, :_HID], jnp.ones((n, 8), jnp.float32)], axis=1)
    batch3 = batch.astype(jnp.int32).reshape(10, 1, 1000)
    return _pool(hcat, batch3, Wc, bc, bm=1000)
